# baseline (speedup 1.0000x reference)
import functools

import jax
import jax.numpy as jnp
from jax import lax
from jax.experimental import pallas as pl
from jax.experimental.pallas import tpu as pltpu
from jax.experimental.pallas import tpu_sc as plsc

NC = 2
NS = 16
NW = NC * NS
PAD = 128
BBLK = 128
NBG = 4


def _mesh():
    return plsc.VectorSubcoreMesh(
        core_axis_name="c", subcore_axis_name="s",
        num_cores=NC, num_subcores=NS)


def _wid():
    return lax.axis_index("s") * NC + lax.axis_index("c")


def _sc_params():
    return pltpu.CompilerParams(use_tc_tiling_on_sc=True,
                                needs_layout_passes=False,
                                disable_bounds_checks=True)


def _build_transpose(emb_dim: int, n_full_cols: int, out_rows: int):
    per_w = n_full_cols // NW
    rem = n_full_cols - per_w * NW
    assert per_w % 2 == 0 and per_w >= 4

    @functools.partial(
        pl.kernel,
        out_type=jax.ShapeDtypeStruct((out_rows, PAD), jnp.float32),
        mesh=_mesh(),
        scratch_types=[
            pltpu.VMEM((2, emb_dim, 128), jnp.float32),
            pltpu.VMEM((2, 128, PAD + 1), jnp.float32),
            pltpu.SemaphoreType.DMA,
            pltpu.SemaphoreType.DMA,
        ],
        compiler_params=_sc_params(),
    )
    def transpose_k(tab_t_hbm, tail_hbm, out_hbm, xb, yb, sem_i, sem_o):
        wid = _wid()
        base = wid * per_w
        n_tail = tail_hbm.shape[0]

        def in_desc(tc, b):
            return pltpu.make_async_copy(
                tab_t_hbm.at[:, pl.ds(tc * 128, 128)], xb.at[b], sem_i)

        def out_desc(tc, b):
            return pltpu.make_async_copy(
                yb.at[b, :, pl.ds(0, PAD)],
                out_hbm.at[pl.ds(tc * 128, 128)], sem_o)

        lanes = lax.iota(jnp.int32, 16)

        def transpose_buf(b):
            ybv = yb.at[b]
            def tbody(i, carry):
                l0 = i * 16
                lvec = lanes + l0
                for d in range(emb_dim):
                    vec = xb[b, d, pl.ds(l0, 16)]
                    plsc.store_scatter(ybv, [lvec, jnp.full((16,), d,
                                                           jnp.int32)], vec)
                return carry
            lax.fori_loop(0, 8, tbody, 0)

        def step(i, b, first, last):
            tc = base + 2 * i + b
            in_desc(tc, b).wait()
            if not first:
                out_desc(tc - 2, b).wait()
            transpose_buf(b)
            out_desc(tc, b).start()
            if not last:
                in_desc(tc + 2, b).start()

        in_desc(base, 0).start()
        in_desc(base + 1, 1).start()
        for b in range(2):
            step(0, b, True, False)

        def middle(i, carry):
            for b in range(2):
                step(i, b, False, False)
            return carry

        lax.fori_loop(1, per_w // 2 - 1, middle, 0)

        for b in range(2):
            step(per_w // 2 - 1, b, False, True)
        for b in range(2):
            out_desc(base + per_w - 2 + b, b).wait()

        @pl.when(wid < rem)
        def _():
            tc = NW * per_w + wid
            pltpu.sync_copy(tab_t_hbm.at[:, pl.ds(tc * 128, 128)], xb.at[0])
            transpose_buf(0)
            pltpu.sync_copy(yb.at[0, :, pl.ds(0, PAD)],
                            out_hbm.at[pl.ds(tc * 128, 128)])

        @pl.when(wid == NW - 1)
        def _():
            pltpu.sync_copy(tail_hbm, xb.at[0, pl.ds(0, n_tail), :])
            pltpu.sync_copy(
                xb.at[0, pl.ds(0, n_tail), :],
                out_hbm.at[pl.ds(n_full_cols * 128, n_tail)])

    return transpose_k


def _build_gather(batch: int, hist: int, emb_dim: int):
    assert batch == NW * BBLK
    assert hist % NBG == 0 and hist >= 3 * NBG

    @functools.partial(
        pl.kernel,
        out_type=jax.ShapeDtypeStruct((hist, emb_dim, batch), jnp.float32),
        mesh=_mesh(),
        scratch_types=[
            pltpu.VMEM((hist, BBLK), jnp.int32),
            pltpu.VMEM((NBG, BBLK, PAD), jnp.float32),
            pltpu.VMEM((2, emb_dim, BBLK + 1), jnp.float32),
            pltpu.SemaphoreType.DMA,
            pltpu.SemaphoreType.DMA,
            pltpu.SemaphoreType.DMA,
        ],
        compiler_params=_sc_params(),
    )
    def emb_gather(table_hbm, idx_hbm, out_hbm, idx_v, gb, yb,
                   sem_i, sem_g, sem_o):
        wid = _wid()
        b0 = wid * BBLK
        pltpu.make_async_copy(idx_hbm.at[:, pl.ds(b0, BBLK)], idx_v,
                              sem_i).start()

        lanes = lax.iota(jnp.int32, 16)

        def gather_desc(h, g):
            return pltpu.make_async_copy(table_hbm.at[idx_v.at[h]],
                                         gb.at[g], sem_g)

        def out_desc(h, y):
            return pltpu.make_async_copy(
                yb.at[y, :, pl.ds(0, BBLK)],
                out_hbm.at[h, :, pl.ds(b0, BBLK)], sem_o)

        ksel = [lanes + 16 * k for k in range(emb_dim // 16)]

        def transpose_buf(g, y):
            ybv = yb.at[y]
            def tbody(j, carry):
                jsplat = jnp.full((16,), j, jnp.int32)
                for k in range(emb_dim // 16):
                    vec = gb[g, j, pl.ds(16 * k, 16)]
                    plsc.store_scatter(ybv, [ksel[k], jsplat], vec)
                return carry
            lax.fori_loop(0, BBLK, tbody, 0)

        pltpu.make_async_copy(idx_hbm.at[:, pl.ds(b0, BBLK)], idx_v,
                              sem_i).wait()
        for g in range(NBG):
            gather_desc(g, g).start()

        def step(h, g, first, last):
            gather_desc(h, g).wait()
            if not (first and g < 2):
                out_desc(h - 2, g % 2).wait()
            transpose_buf(g, g % 2)
            out_desc(h, g % 2).start()
            if not last:
                gather_desc(h + NBG, g).start()

        for g in range(NBG):
            step(g, g, True, False)

        def middle(i, carry):
            h0 = i * NBG
            for g in range(NBG):
                step(h0 + g, g, False, False)
            return carry

        lax.fori_loop(1, hist // NBG - 1, middle, 0)

        for g in range(NBG):
            step(hist - NBG + g, g, False, True)

        out_desc(hist - 2, (hist - 2) % 2).wait()
        out_desc(hist - 1, (hist - 1) % 2).wait()

    return emb_gather


def kernel(batch_input, lengths, embedding_table):
    del lengths
    batch, hist = batch_input.shape
    vocab, emb_dim = embedding_table.shape

    n_full_cols = vocab // 128
    v_main = n_full_cols * 128
    out_rows = (n_full_cols + 1) * 128

    tab_t = embedding_table.T
    tail = jnp.pad(embedding_table[v_main:, :],
                   ((0, 0), (0, PAD - emb_dim)))
    idx_t = batch_input.T.astype(jnp.int32)

    table_pad = _build_transpose(emb_dim, n_full_cols, out_rows)(tab_t, tail)
    out = _build_gather(batch, hist, emb_dim)(table_pad, idx_t)
    return out.transpose(2, 0, 1)


# speedup vs baseline: 2.2079x; 2.2079x over previous
import functools

import jax
import jax.numpy as jnp
from jax import lax
from jax.experimental import pallas as pl
from jax.experimental.pallas import tpu as pltpu
from jax.experimental.pallas import tpu_sc as plsc

NC = 2
NS = 16
NW = NC * NS
CHUNK = 128
NBUF = 5
PRIME = 4
LAG = NBUF - PRIME
PAD = 128


def _build_gather(total: int, emb_dim: int):
    n_chunks = total // (NW * CHUNK)
    assert n_chunks * NW * CHUNK == total
    assert n_chunks % NBUF == 0 and n_chunks >= 2 * NBUF
    n_outer = n_chunks // NBUF
    rows_per_worker = n_chunks * CHUNK

    mesh = plsc.VectorSubcoreMesh(
        core_axis_name="c", subcore_axis_name="s",
        num_cores=NC, num_subcores=NS)

    @functools.partial(
        pl.kernel,
        out_type=jax.ShapeDtypeStruct((total, PAD), jnp.float32),
        mesh=mesh,
        scratch_types=[
            pltpu.VMEM((n_chunks, CHUNK), jnp.int32),
            pltpu.VMEM((NBUF, CHUNK, PAD), jnp.float32),
            pltpu.SemaphoreType.DMA,
            pltpu.SemaphoreType.DMA,
        ],
        compiler_params=pltpu.CompilerParams(use_tc_tiling_on_sc=True),
    )
    def emb_gather(table_hbm, idx_hbm, out_hbm, idx_v, bufs, sem_g, sem_o):
        wid = lax.axis_index("s") * NC + lax.axis_index("c")
        base = wid * rows_per_worker
        pltpu.sync_copy(idx_hbm.at[wid], idx_v)

        def gather_desc(g, b):
            return pltpu.make_async_copy(table_hbm.at[idx_v.at[g]],
                                         bufs.at[b], sem_g)

        def scatter_desc(g, b):
            return pltpu.make_async_copy(
                bufs.at[b],
                out_hbm.at[pl.ds(base + g * CHUNK, CHUNK)], sem_o)

        for b in range(PRIME):
            gather_desc(b, b).start()

        def step(t, b, first, last):
            gather_desc(t, b).wait()
            scatter_desc(t, b).start()
            if first and b < LAG:
                gather_desc(t + PRIME, (b + PRIME) % NBUF).start()
            elif not (last and b >= LAG):
                scatter_desc(t - LAG, (b - LAG) % NBUF).wait()
                gather_desc(t + PRIME, (b + PRIME) % NBUF).start()
            else:
                scatter_desc(t - LAG, (b - LAG) % NBUF).wait()

        for b in range(NBUF):
            step(b, b, True, False)

        def outer(i, carry):
            t0 = i * NBUF
            for b in range(NBUF):
                step(t0 + b, b, False, False)
            return carry

        lax.fori_loop(1, n_outer - 1, outer, 0)

        for b in range(NBUF):
            step(n_chunks - NBUF + b, b, False, True)

        for k in range(LAG):
            g = n_chunks - LAG + k
            scatter_desc(g, g % NBUF).wait()

    return emb_gather


def kernel(batch_input, lengths, embedding_table):
    del lengths
    batch, hist = batch_input.shape
    vocab, emb_dim = embedding_table.shape
    total = batch * hist
    n_chunks = total // (NW * CHUNK)
    idx = batch_input.reshape(NW, n_chunks, CHUNK).astype(jnp.int32)
    table_padded = jnp.pad(embedding_table, ((0, 0), (0, PAD - emb_dim)))
    out = _build_gather(total, emb_dim)(table_padded, idx)
    return out[:, :emb_dim].reshape(batch, hist, emb_dim)
